# baseline (speedup 1.0000x reference)
import jax
import jax.numpy as jnp
from jax import lax
from jax.experimental import pallas as pl
from jax.experimental.pallas import tpu as pltpu
from jax.experimental.pallas import tpu_sc as plsc

_C = 256
_HW = 4096
_N = 32768
_NC = 2
_NS = 16
_L = 16
_PKC = _C // 2
_CPT = _PKC // _NS
_NSAMP = _N // _NC
_CH = 2048
_NCHUNK = _NSAMP // _CH
_GPC = _CH // _L


def _sc_body(fmr_hbm, sr_hbm, ii_hbm, ij_hbm, out_hbm,
             tblf_v, tbls_v, ii_v, ip_v, jj_v, jp_v,
             pacc_v, isem, osem):
    cid = lax.axis_index("c")
    sid = lax.axis_index("s")
    sbase = cid * _NSAMP

    t1 = pltpu.async_copy(fmr_hbm.at[pl.ds(sid * _CPT, _CPT)], tblf_v, osem.at[0])
    t2 = pltpu.async_copy(sr_hbm.at[pl.ds(sid * _CPT, _CPT)], tbls_v, osem.at[1])
    t1.wait()
    t2.wait()

    bzero = jnp.zeros((2 * _L,), jnp.bfloat16)

    def fetch_idx(ch, par):
        off = sbase + ch * _CH
        sem = isem.at[par]
        pltpu.async_copy(ii_hbm.at[pl.ds(off, _CH)], ii_v.at[par], sem)
        pltpu.async_copy(ii_hbm.at[pl.ds(_N + off, _CH)], ip_v.at[par], sem)
        pltpu.async_copy(ij_hbm.at[pl.ds(off, _CH)], jj_v.at[par], sem)
        pltpu.async_copy(ij_hbm.at[pl.ds(_N + off, _CH)], jp_v.at[par], sem)

    def drain_idx(par):
        sem = isem.at[par]
        dummy = ii_hbm.at[pl.ds(0, _CH)]
        pltpu.make_async_copy(dummy, ii_v.at[par], sem).wait()
        pltpu.make_async_copy(dummy, ip_v.at[par], sem).wait()
        pltpu.make_async_copy(dummy, jj_v.at[par], sem).wait()
        pltpu.make_async_copy(dummy, jp_v.at[par], sem).wait()

    def put_out(ch, par):
        sem = osem.at[par]
        cols = pl.ds(sbase + ch * _CH, _CH)
        for role in range(4):
            pltpu.async_copy(pacc_v.at[par, role],
                             out_hbm.at[role, sid, cols], sem)

    def drain_out(par):
        sem = osem.at[par]
        dummy = out_hbm.at[0, 0, pl.ds(0, _CH)]
        for role in range(4):
            pltpu.make_async_copy(dummy, pacc_v.at[par, role], sem).wait()

    fetch_idx(0, 0)
    fetch_idx(1, 1)

    def chunk_loop(ch, _):
        par = lax.rem(ch, 2)
        drain_idx(par)

        @pl.when(ch >= 2)
        def _():
            drain_out(par)

        iir = ii_v.at[par]
        ipr = ip_v.at[par]
        jjr = jj_v.at[par]
        jpr = jp_v.at[par]
        pa0 = pacc_v.at[par, 0]
        pa1 = pacc_v.at[par, 1]
        pa2 = pacc_v.at[par, 2]
        pa3 = pacc_v.at[par, 3]

        @plsc.parallel_loop(0, _GPC, 1, unroll=2)
        def group(g):
            gb = g * _L
            i_vec = iir[pl.ds(gb, _L)]
            ip_vec = ipr[pl.ds(gb, _L)]
            j_vec = jjr[pl.ds(gb, _L)]
            jp_vec = jpr[pl.ds(gb, _L)]
            p1 = bzero
            p2 = bzero
            p3 = bzero
            p4 = bzero
            for c in range(_CPT):
                a = plsc.bitcast(plsc.load_gather(tblf_v.at[c], [i_vec]),
                                 jnp.bfloat16)
                b = plsc.bitcast(plsc.load_gather(tblf_v.at[c], [ip_vec]),
                                 jnp.bfloat16)
                cs = plsc.bitcast(plsc.load_gather(tbls_v.at[c], [j_vec]),
                                  jnp.bfloat16)
                ds_ = plsc.bitcast(plsc.load_gather(tbls_v.at[c], [jp_vec]),
                                   jnp.bfloat16)
                p1 = p1 + a * b
                p2 = p2 + cs * ds_
                p3 = p3 + a * ds_
                p4 = p4 + cs * b
            u0, u1 = plsc.unpack(p1, format=plsc.PackFormat.INTERLEAVED)
            pa0[pl.ds(gb, _L)] = u0 + u1
            u0, u1 = plsc.unpack(p2, format=plsc.PackFormat.INTERLEAVED)
            pa1[pl.ds(gb, _L)] = u0 + u1
            u0, u1 = plsc.unpack(p3, format=plsc.PackFormat.INTERLEAVED)
            pa2[pl.ds(gb, _L)] = u0 + u1
            u0, u1 = plsc.unpack(p4, format=plsc.PackFormat.INTERLEAVED)
            pa3[pl.ds(gb, _L)] = u0 + u1
        put_out(ch, par)

        @pl.when(ch < _NCHUNK - 2)
        def _():
            fetch_idx(ch + 2, par)

        return 0

    lax.fori_loop(0, _NCHUNK, chunk_loop, 0)
    drain_out(0)
    drain_out(1)


def _poly_loss_sc(fmr, sr, idx_i, idx_j):
    mesh = plsc.VectorSubcoreMesh(core_axis_name="c", subcore_axis_name="s")
    call = pl.kernel(
        _sc_body,
        out_type=jax.ShapeDtypeStruct((4, _NS, _N), jnp.float32),
        mesh=mesh,
        scratch_types=[
            pltpu.VMEM((_CPT, _HW), jnp.int32),
            pltpu.VMEM((_CPT, _HW), jnp.int32),
            pltpu.VMEM((2, _CH), jnp.int32),
            pltpu.VMEM((2, _CH), jnp.int32),
            pltpu.VMEM((2, _CH), jnp.int32),
            pltpu.VMEM((2, _CH), jnp.int32),
            pltpu.VMEM((2, 4, _CH), jnp.float32),
            pltpu.SemaphoreType.DMA((2,)),
            pltpu.SemaphoreType.DMA((2,)),
        ],
        compiler_params=pltpu.CompilerParams(
            needs_layout_passes=False,
            use_tc_tiling_on_sc=False,
            disable_bounds_checks=True,
        ),
    )
    return call(fmr, sr, idx_i, idx_j)


_TCBLK = 8192


def _combine_body(p_ref, o_ref):
    k = pl.program_id(0)
    x = p_ref[...]
    s = jnp.sum(x, axis=1)
    q = s * s
    psum = jnp.sum(q[0:2]) - jnp.sum(q[2:4])

    @pl.when(k == 0)
    def _():
        o_ref[0, 0] = psum

    @pl.when(k != 0)
    def _():
        o_ref[0, 0] += psum


def _combine(p):
    grid = _N // _TCBLK
    return pl.pallas_call(
        _combine_body,
        grid=(grid,),
        in_specs=[pl.BlockSpec((4, _NS, _TCBLK), lambda k: (0, 0, k))],
        out_specs=pl.BlockSpec(memory_space=pltpu.SMEM),
        out_shape=jax.ShapeDtypeStruct((1, 1), jnp.float32),
    )(p)


def _pack_table(x, c):
    u = jax.lax.bitcast_convert_type(x.astype(jnp.bfloat16), jnp.uint16)
    lo = u[0::2].astype(jnp.uint32)
    hi = u[1::2].astype(jnp.uint32)
    return jax.lax.bitcast_convert_type(lo | (hi << 16), jnp.int32)


def kernel(input, target, idx_i, idx_j):
    c = input.shape[1]
    fmr = _pack_table(input.reshape(c, -1), c)
    sr = _pack_table(target.reshape(c, -1), c)
    p = _poly_loss_sc(fmr, sr, idx_i, idx_j)
    total = _combine(p)
    n = idx_i.shape[0] // 2
    return total[0, 0] / jnp.float32(n) / jnp.float32(c * c)


# speedup vs baseline: 4.1650x; 4.1650x over previous
import jax
import jax.numpy as jnp
from jax import lax
from jax.experimental import pallas as pl
from jax.experimental.pallas import tpu as pltpu
from jax.experimental.pallas import tpu_sc as plsc

_C = 256
_HW = 4096
_N = 32768
_NC = 2
_NS = 16
_L = 16
_PKC = _C // 2
_CPT = _PKC // _NS
_NSAMP = _N // _NC
_CH = 2048
_NCHUNK = _NSAMP // _CH
_GPC = _CH // _L


def _sc_body(fmr_hbm, sr_hbm, ii_hbm, ij_hbm, out_hbm,
             tblf_v, tbls_v, ii_v, ip_v, jj_v, jp_v,
             pacc_v, isem, osem):
    cid = lax.axis_index("c")
    sid = lax.axis_index("s")
    sbase = cid * _NSAMP

    t1 = pltpu.async_copy(fmr_hbm.at[pl.ds(sid * _CPT, _CPT)], tblf_v, osem.at[0])
    t2 = pltpu.async_copy(sr_hbm.at[pl.ds(sid * _CPT, _CPT)], tbls_v, osem.at[1])
    t1.wait()
    t2.wait()

    bzero = jnp.zeros((2 * _L,), jnp.bfloat16)

    def fetch_idx(ch, par):
        off = sbase + ch * _CH
        sem = isem.at[par]
        pltpu.async_copy(ii_hbm.at[pl.ds(off, _CH)], ii_v.at[par], sem)
        pltpu.async_copy(ii_hbm.at[pl.ds(_N + off, _CH)], ip_v.at[par], sem)
        pltpu.async_copy(ij_hbm.at[pl.ds(off, _CH)], jj_v.at[par], sem)
        pltpu.async_copy(ij_hbm.at[pl.ds(_N + off, _CH)], jp_v.at[par], sem)

    def drain_idx(par):
        sem = isem.at[par]
        dummy = ii_hbm.at[pl.ds(0, _CH)]
        pltpu.make_async_copy(dummy, ii_v.at[par], sem).wait()
        pltpu.make_async_copy(dummy, ip_v.at[par], sem).wait()
        pltpu.make_async_copy(dummy, jj_v.at[par], sem).wait()
        pltpu.make_async_copy(dummy, jp_v.at[par], sem).wait()

    def put_out(ch, par):
        sem = osem.at[par]
        cols = pl.ds(sbase + ch * _CH, _CH)
        for role in range(4):
            pltpu.async_copy(pacc_v.at[par, role],
                             out_hbm.at[role, sid, cols], sem)

    def drain_out(par):
        sem = osem.at[par]
        dummy = out_hbm.at[0, 0, pl.ds(0, _CH)]
        for role in range(4):
            pltpu.make_async_copy(dummy, pacc_v.at[par, role], sem).wait()

    fetch_idx(0, 0)
    fetch_idx(1, 1)

    def chunk_loop(ch, _):
        par = lax.rem(ch, 2)
        drain_idx(par)

        @pl.when(ch >= 2)
        def _():
            drain_out(par)

        iir = ii_v.at[par]
        ipr = ip_v.at[par]
        jjr = jj_v.at[par]
        jpr = jp_v.at[par]
        pa0 = pacc_v.at[par, 0]
        pa1 = pacc_v.at[par, 1]
        pa2 = pacc_v.at[par, 2]
        pa3 = pacc_v.at[par, 3]

        @plsc.parallel_loop(0, _GPC, 1, unroll=2)
        def group(g):
            gb = g * _L
            i_vec = iir[pl.ds(gb, _L)]
            ip_vec = ipr[pl.ds(gb, _L)]
            j_vec = jjr[pl.ds(gb, _L)]
            jp_vec = jpr[pl.ds(gb, _L)]
            p1 = bzero
            p2 = bzero
            p3 = bzero
            p4 = bzero
            for c in range(_CPT):
                a = plsc.bitcast(plsc.load_gather(tblf_v.at[c], [i_vec]),
                                 jnp.bfloat16)
                b = plsc.bitcast(plsc.load_gather(tblf_v.at[c], [ip_vec]),
                                 jnp.bfloat16)
                cs = plsc.bitcast(plsc.load_gather(tbls_v.at[c], [j_vec]),
                                  jnp.bfloat16)
                ds_ = plsc.bitcast(plsc.load_gather(tbls_v.at[c], [jp_vec]),
                                   jnp.bfloat16)
                p1 = p1 + a * b
                p2 = p2 + cs * ds_
                p3 = p3 + a * ds_
                p4 = p4 + cs * b
            u0, u1 = plsc.unpack(p1, format=plsc.PackFormat.INTERLEAVED)
            pa0[pl.ds(gb, _L)] = u0 + u1
            u0, u1 = plsc.unpack(p2, format=plsc.PackFormat.INTERLEAVED)
            pa1[pl.ds(gb, _L)] = u0 + u1
            u0, u1 = plsc.unpack(p3, format=plsc.PackFormat.INTERLEAVED)
            pa2[pl.ds(gb, _L)] = u0 + u1
            u0, u1 = plsc.unpack(p4, format=plsc.PackFormat.INTERLEAVED)
            pa3[pl.ds(gb, _L)] = u0 + u1
        put_out(ch, par)

        @pl.when(ch < _NCHUNK - 2)
        def _():
            fetch_idx(ch + 2, par)

        return 0

    lax.fori_loop(0, _NCHUNK, chunk_loop, 0)
    drain_out(0)
    drain_out(1)


def _poly_loss_sc(fmr, sr, idx_i, idx_j):
    mesh = plsc.VectorSubcoreMesh(core_axis_name="c", subcore_axis_name="s")
    call = pl.kernel(
        _sc_body,
        out_type=jax.ShapeDtypeStruct((4, _NS, _N), jnp.float32),
        mesh=mesh,
        scratch_types=[
            pltpu.VMEM((_CPT, _HW), jnp.int32),
            pltpu.VMEM((_CPT, _HW), jnp.int32),
            pltpu.VMEM((2, _CH), jnp.int32),
            pltpu.VMEM((2, _CH), jnp.int32),
            pltpu.VMEM((2, _CH), jnp.int32),
            pltpu.VMEM((2, _CH), jnp.int32),
            pltpu.VMEM((2, 4, _CH), jnp.float32),
            pltpu.SemaphoreType.DMA((2,)),
            pltpu.SemaphoreType.DMA((2,)),
        ],
        compiler_params=pltpu.CompilerParams(
            needs_layout_passes=False,
            use_tc_tiling_on_sc=False,
            disable_bounds_checks=True,
        ),
    )
    return call(fmr, sr, idx_i, idx_j)


_TCBLK = 8192


def _combine_body(p_ref, o_ref):
    k = pl.program_id(0)
    x = p_ref[...]
    s = jnp.sum(x, axis=1)
    q = s * s
    psum = jnp.sum(q[0:2]) - jnp.sum(q[2:4])

    @pl.when(k == 0)
    def _():
        o_ref[0, 0] = psum

    @pl.when(k != 0)
    def _():
        o_ref[0, 0] += psum


def _combine(p):
    grid = _N // _TCBLK
    return pl.pallas_call(
        _combine_body,
        grid=(grid,),
        in_specs=[pl.BlockSpec((4, _NS, _TCBLK), lambda k: (0, 0, k))],
        out_specs=pl.BlockSpec(memory_space=pltpu.SMEM),
        out_shape=jax.ShapeDtypeStruct((1, 1), jnp.float32),
    )(p)


def _pack_table(x, c):
    u = jax.lax.bitcast_convert_type(x.astype(jnp.bfloat16), jnp.uint16)
    lo = u[: c // 2].astype(jnp.uint32)
    hi = u[c // 2:].astype(jnp.uint32)
    return jax.lax.bitcast_convert_type(lo | (hi << 16), jnp.int32)


def kernel(input, target, idx_i, idx_j):
    c = input.shape[1]
    fmr = _pack_table(input.reshape(c, -1), c)
    sr = _pack_table(target.reshape(c, -1), c)
    p = _poly_loss_sc(fmr, sr, idx_i, idx_j)
    total = _combine(p)
    n = idx_i.shape[0] // 2
    return total[0, 0] / jnp.float32(n) / jnp.float32(c * c)
